# baseline (speedup 1.0000x reference)
import functools

import jax
import jax.numpy as jnp
from jax import lax
from jax.experimental import pallas as pl
from jax.experimental.pallas import tpu as pltpu
from jax.experimental.pallas import tpu_sc as plsc

N = 10000
NPAD = 10240
E = 320000
G = 128
NC = 2
NS = 16
NW = NC * NS
CH = 128
KC = 80
EPAD = NW * KC * CH
NBUF = 2
NLAG = 1
KC2 = 2 * KC
ASPL = 78
MAXC = max(ASPL, KC2 - ASPL)
RPS = NPAD // NS
BLK = 2560
NG = NPAD // BLK
_EPS = 1e-5

_mesh = plsc.VectorSubcoreMesh(core_axis_name="c", subcore_axis_name="s")


def _degree_body(src_hbm, dst_hbm, eye_hbm, z8_hbm, deg_hbm,
                 src_v, dst_v, e0_v, e1_v, deg_sh, sem, sem2):
    cid = lax.axis_index("c")
    sid = lax.axis_index("s")
    wid = sid * NC + cid
    pltpu.sync_copy(src_hbm.at[wid], src_v)
    pltpu.sync_copy(dst_hbm.at[wid], dst_v)
    pltpu.sync_copy(eye_hbm.at[0], e0_v)
    pltpu.sync_copy(eye_hbm.at[1], e1_v)
    pltpu.sync_copy(z8_hbm.at[pl.ds(sid * RPS, RPS)],
                    deg_sh.at[pl.ds(sid * RPS, RPS)])
    plsc.subcore_barrier()

    def body(j, carry):
        pltpu.async_copy(e0_v, deg_sh.at[src_v.at[j]], sem, add=True)
        pltpu.async_copy(e1_v, deg_sh.at[dst_v.at[j]], sem2, add=True)

        @pl.when(j >= 1)
        def _():
            pltpu.make_async_copy(e0_v, deg_sh.at[src_v.at[j - 1]],
                                  sem).wait()
            pltpu.make_async_copy(e1_v, deg_sh.at[dst_v.at[j - 1]],
                                  sem2).wait()

        return carry

    lax.fori_loop(0, KC, body, 0)
    pltpu.make_async_copy(e0_v, deg_sh.at[src_v.at[KC - 1]], sem).wait()
    pltpu.make_async_copy(e1_v, deg_sh.at[dst_v.at[KC - 1]], sem2).wait()
    plsc.subcore_barrier()
    pltpu.sync_copy(deg_sh.at[pl.ds(sid * RPS, RPS)],
                    deg_hbm.at[cid].at[pl.ds(sid * RPS, RPS)])


_degree = functools.partial(
    pl.kernel,
    out_type=jax.ShapeDtypeStruct((NC, NPAD, 8), jnp.float32),
    mesh=_mesh,
    scratch_types=[
        pltpu.VMEM((KC, CH), jnp.int32),
        pltpu.VMEM((KC, CH), jnp.int32),
        pltpu.VMEM((CH, 8), jnp.float32),
        pltpu.VMEM((CH, 8), jnp.float32),
        pltpu.VMEM_SHARED((NPAD, 8), jnp.float32),
        pltpu.SemaphoreType.DMA,
        pltpu.SemaphoreType.DMA,
    ],
    compiler_params=pltpu.CompilerParams(use_tc_tiling_on_sc=False),
)(_degree_body)


def _make_edge_pass(D, nslab):
    def body(*refs):
        xs_list = refs[:nslab]
        src_hbm, dst_hbm, zd_hbm = refs[nslab:nslab + 3]
        out_list = refs[nslab + 3:nslab + 3 + nslab]
        (src_v, dst_v, r0, r1, xs_sh, agg_sh,
         g0, g1, s0, s1) = refs[nslab + 3 + nslab:]
        cid = lax.axis_index("c")
        sid = lax.axis_index("s")
        rows = (r0, r1)
        gsem = (g0, g1)
        ssem = (s0, s1)

        def run(off, cnt, load_idx):
            if load_idx:
                pltpu.sync_copy(src_hbm.at[sid].at[pl.ds(off, cnt)],
                                src_v.at[pl.ds(0, cnt)])
                pltpu.sync_copy(dst_hbm.at[sid].at[pl.ds(off, cnt)],
                                dst_v.at[pl.ds(0, cnt)])
            for b in range(NLAG):
                pltpu.async_copy(xs_sh.at[src_v.at[b]], rows[b], gsem[b])

            def group(gi, carry):
                for b in range(NBUF):
                    j = gi * NBUF + b
                    bh = (b + NLAG) % NBUF
                    pltpu.make_async_copy(xs_sh.at[src_v.at[j]], rows[b],
                                          gsem[b]).wait()
                    pltpu.async_copy(rows[b], agg_sh.at[dst_v.at[j]],
                                     ssem[b], add=True)

                    @pl.when(j >= NLAG)
                    def _():
                        pltpu.make_async_copy(rows[bh],
                                              agg_sh.at[dst_v.at[j - NLAG]],
                                              ssem[bh]).wait()

                    @pl.when(j + NLAG < cnt)
                    def _():
                        pltpu.async_copy(xs_sh.at[src_v.at[j + NLAG]],
                                         rows[bh], gsem[bh])
                return carry

            lax.fori_loop(0, cnt // NBUF, group, 0)
            for c in range(cnt - NLAG, cnt):
                pltpu.make_async_copy(rows[c % NBUF], agg_sh.at[dst_v.at[c]],
                                      ssem[c % NBUF]).wait()

        for k in range(nslab):
            pltpu.sync_copy(xs_list[k].at[pl.ds(sid * RPS, RPS)],
                            xs_sh.at[pl.ds(sid * RPS, RPS)])
            pltpu.sync_copy(zd_hbm.at[pl.ds(sid * RPS, RPS)],
                            agg_sh.at[pl.ds(sid * RPS, RPS)])
            plsc.subcore_barrier()

            @pl.when(cid == 0)
            def _():
                run(0, ASPL, k == 0)

            @pl.when(cid == 1)
            def _():
                run(ASPL, KC2 - ASPL, k == 0)

            plsc.subcore_barrier()
            pltpu.sync_copy(agg_sh.at[pl.ds(sid * RPS, RPS)],
                            out_list[k].at[cid].at[pl.ds(sid * RPS, RPS)])

    return functools.partial(
        pl.kernel,
        out_type=[jax.ShapeDtypeStruct((NC, NPAD, D), jnp.float32)
                  for _ in range(nslab)],
        mesh=_mesh,
        scratch_types=[
            pltpu.VMEM((MAXC, CH), jnp.int32),
            pltpu.VMEM((MAXC, CH), jnp.int32),
        ] + [pltpu.VMEM((CH, D), jnp.float32) for _ in range(NBUF)] + [
            pltpu.VMEM_SHARED((NPAD, D), jnp.float32),
            pltpu.VMEM_SHARED((NPAD, D), jnp.float32),
        ] + [pltpu.SemaphoreType.DMA for _ in range(2 * NBUF)],
        compiler_params=pltpu.CompilerParams(use_tc_tiling_on_sc=False),
    )(body)


_edge64 = _make_edge_pass(64, 1)


def _prep_body(deg_ref, x_ref, no_ref, ni_ref, xsa_ref, xsb_ref):
    d = deg_ref[0] + deg_ref[1]
    do = d[:, 0:1]
    di = d[:, 1:2]
    no = jnp.where(do > 0, lax.rsqrt(jnp.maximum(do, 1.0)), 0.0)
    ni = jnp.where(di > 0, lax.rsqrt(jnp.maximum(di, 1.0)), 0.0)
    no_ref[...] = jnp.broadcast_to(no, (BLK, 8))
    ni_ref[...] = jnp.broadcast_to(ni, (BLK, 8))
    xs = x_ref[...] * no
    xsa_ref[...] = xs[:, :64]
    xsb_ref[...] = xs[:, 64:]


def _prep(degp, xp):
    return pl.pallas_call(
        _prep_body,
        grid=(NG,),
        in_specs=[
            pl.BlockSpec((NC, BLK, 8), lambda i: (0, i, 0)),
            pl.BlockSpec((BLK, 128), lambda i: (i, 0)),
        ],
        out_specs=[
            pl.BlockSpec((BLK, 8), lambda i: (i, 0)),
            pl.BlockSpec((BLK, 8), lambda i: (i, 0)),
            pl.BlockSpec((BLK, 64), lambda i: (i, 0)),
            pl.BlockSpec((BLK, 64), lambda i: (i, 0)),
        ],
        out_shape=[
            jax.ShapeDtypeStruct((NPAD, 8), jnp.float32),
            jax.ShapeDtypeStruct((NPAD, 8), jnp.float32),
            jax.ShapeDtypeStruct((NPAD, 64), jnp.float32),
            jax.ShapeDtypeStruct((NPAD, 64), jnp.float32),
        ],
    )(degp, xp)


def _make_layer_body(slabs):
    def body(*refs):
        p_refs = refs[:slabs]
        (x_ref, ni_ref, no_ref, w_ref, b_ref, rw_ref, rb_ref,
         g_ref, beta_ref, m_ref, v_ref, h_ref, hs_ref) = refs[slabs:]
        ni = ni_ref[...][:, 0:1]
        aggs = [(p[0] + p[1]) * ni for p in p_refs]
        agg = aggs[0] if slabs == 1 else jnp.concatenate(aggs, axis=1)
        hm = jnp.dot(agg, w_ref[...], preferred_element_type=jnp.float32)
        hm = jnp.maximum(hm + b_ref[...], 0.0)
        res = jnp.dot(x_ref[...], rw_ref[...],
                      preferred_element_type=jnp.float32)
        res = jnp.maximum(res + rb_ref[...], 0.0)
        h = hm + res
        h = (h - m_ref[...]) * lax.rsqrt(v_ref[...] + _EPS) * g_ref[...] \
            + beta_ref[...]
        h_ref[...] = h
        hs_ref[...] = h * no_ref[...][:, 0:1]
    return body


def _layer(p_list, x, ni, no, w, b, rw, rb, g, beta, m, v):
    slabs = len(p_list)
    D = 64 * slabs
    vec = lambda: pl.BlockSpec((1, 64), lambda i: (0, 0))
    return pl.pallas_call(
        _make_layer_body(slabs),
        grid=(NG,),
        in_specs=[pl.BlockSpec((NC, BLK, 64), lambda i: (0, i, 0))
                  for _ in range(slabs)] + [
            pl.BlockSpec((BLK, D), lambda i: (i, 0)),
            pl.BlockSpec((BLK, 8), lambda i: (i, 0)),
            pl.BlockSpec((BLK, 8), lambda i: (i, 0)),
            pl.BlockSpec((D, 64), lambda i: (0, 0)),
            vec(),
            pl.BlockSpec((D, 64), lambda i: (0, 0)),
            vec(), vec(), vec(), vec(), vec(),
        ],
        out_specs=[
            pl.BlockSpec((BLK, 64), lambda i: (i, 0)),
            pl.BlockSpec((BLK, 64), lambda i: (i, 0)),
        ],
        out_shape=[
            jax.ShapeDtypeStruct((NPAD, 64), jnp.float32),
            jax.ShapeDtypeStruct((NPAD, 64), jnp.float32),
        ],
    )(*p_list, x, ni, no, w, b.reshape(1, 64), rw, rb.reshape(1, 64),
      g.reshape(1, 64), beta.reshape(1, 64), m.reshape(1, 64),
      v.reshape(1, 64))


def _l3f_body(p_ref, x_ref, ni_ref, gid_ref, w_ref, b_ref, rw_ref, rb_ref,
              g_ref, beta_ref, m_ref, v_ref, w1_ref, b1_ref, w2_ref, b2_ref,
              out_ref, acc_ref):
    i = pl.program_id(0)

    @pl.when(i == 0)
    def _():
        acc_ref[...] = jnp.zeros_like(acc_ref)
        out_ref[...] = jnp.zeros_like(out_ref)

    agg = (p_ref[0] + p_ref[1]) * ni_ref[...][:, 0:1]
    hm = jnp.dot(agg, w_ref[...], preferred_element_type=jnp.float32)
    hm = jnp.maximum(hm + b_ref[...], 0.0)
    res = jnp.dot(x_ref[...], rw_ref[...], preferred_element_type=jnp.float32)
    res = jnp.maximum(res + rb_ref[...], 0.0)
    h = hm + res
    h = (h - m_ref[...]) * lax.rsqrt(v_ref[...] + _EPS) * g_ref[...] \
        + beta_ref[...]
    gid = gid_ref[0, 0]
    oh = (lax.broadcasted_iota(jnp.int32, (G, BLK), 0) == gid[None, :])
    acc_ref[...] += jnp.dot(oh.astype(jnp.float32), h,
                            preferred_element_type=jnp.float32)

    @pl.when(i == NG - 1)
    def _():
        hid = jnp.dot(acc_ref[...], w1_ref[...],
                      preferred_element_type=jnp.float32)
        hid = jnp.maximum(hid + b1_ref[...], 0.0)
        out_ref[...] = jnp.dot(hid, w2_ref[...],
                               preferred_element_type=jnp.float32) \
            + b2_ref[...]


def _layer3_final(p, x, ni, gid3, w, b, rw, rb, g, beta, m, v,
                  w1, b1, w2p, b2p):
    vec = lambda: pl.BlockSpec((1, 64), lambda i: (0, 0))
    return pl.pallas_call(
        _l3f_body,
        grid=(NG,),
        in_specs=[
            pl.BlockSpec((NC, BLK, 64), lambda i: (0, i, 0)),
            pl.BlockSpec((BLK, 64), lambda i: (i, 0)),
            pl.BlockSpec((BLK, 8), lambda i: (i, 0)),
            pl.BlockSpec((1, 1, BLK), lambda i: (i, 0, 0)),
            pl.BlockSpec((64, 64), lambda i: (0, 0)),
            vec(),
            pl.BlockSpec((64, 64), lambda i: (0, 0)),
            vec(), vec(), vec(), vec(), vec(),
            pl.BlockSpec((64, 128), lambda i: (0, 0)),
            pl.BlockSpec((1, 128), lambda i: (0, 0)),
            pl.BlockSpec((128, 128), lambda i: (0, 0)),
            pl.BlockSpec((1, 128), lambda i: (0, 0)),
        ],
        out_specs=pl.BlockSpec((G, 128), lambda i: (0, 0)),
        out_shape=jax.ShapeDtypeStruct((G, 128), jnp.float32),
        scratch_shapes=[pltpu.VMEM((G, 64), jnp.float32)],
    )(p, x, ni, gid3, w, b.reshape(1, 64), rw, rb.reshape(1, 64),
      g.reshape(1, 64), beta.reshape(1, 64), m.reshape(1, 64),
      v.reshape(1, 64), w1, b1, w2p, b2p)


def kernel(node_feats, edge_index, graph_ids,
           W0, b0, Rw0, Rb0, g0, beta0, m0, v0,
           W1, b1, Rw1, Rb1, g1, beta1, m1, v1,
           W2, b2, Rw2, Rb2, g2, beta2, m2, v2,
           Wc1, bc1, Wc2, bc2):
    pad_e = EPAD - E
    srcp = jnp.concatenate(
        [edge_index[0], jnp.full((pad_e,), N, jnp.int32)]).reshape(NW, KC, CH)
    dstp = jnp.concatenate(
        [edge_index[1], jnp.full((pad_e,), N, jnp.int32)]).reshape(NW, KC, CH)
    xp = jnp.pad(node_feats, ((0, NPAD - N), (0, 0)))
    gid3 = jnp.pad(graph_ids, (0, NPAD - N),
                   constant_values=G).reshape(NG, 1, BLK)
    eye = jnp.zeros((2, CH, 8), jnp.float32)
    eye = eye.at[0, :, 0].set(1.0).at[1, :, 1].set(1.0)
    z8 = jnp.zeros((NPAD, 8), jnp.float32)
    z64 = jnp.zeros((NPAD, 64), jnp.float32)

    degp = _degree(srcp, dstp, eye, z8)
    no, ni, xs0a, xs0b = _prep(degp, xp)

    srcE = srcp.reshape(NS, KC2, CH)
    dstE = dstp.reshape(NS, KC2, CH)
    (p0a,) = _edge64(xs0a, srcE, dstE, z64)
    (p0b,) = _edge64(xs0b, srcE, dstE, z64)
    h1, h1s = _layer([p0a, p0b], xp, ni, no,
                     W0, b0, Rw0, Rb0, g0, beta0, m0, v0)

    (p1,) = _edge64(h1s, srcE, dstE, z64)
    h2, h2s = _layer([p1], h1, ni, no, W1, b1, Rw1, Rb1, g1, beta1, m1, v1)

    (p2,) = _edge64(h2s, srcE, dstE, z64)
    w2p = jnp.pad(Wc2, ((0, 0), (0, 126)))
    b2p = jnp.pad(bc2, (0, 126)).reshape(1, 128)
    logits = _layer3_final(p2, h2, ni, gid3, W2, b2, Rw2, Rb2, g2, beta2,
                           m2, v2, Wc1, bc1.reshape(1, 128), w2p, b2p)
    return logits[:, :2]


# speedup vs baseline: 1.0138x; 1.0138x over previous
import functools

import jax
import jax.numpy as jnp
from jax import lax
from jax.experimental import pallas as pl
from jax.experimental.pallas import tpu as pltpu
from jax.experimental.pallas import tpu_sc as plsc

N = 10000
NPAD = 10240
E = 320000
G = 128
NC = 2
NS = 16
NW = NC * NS
CH = 128
KC = 80
EPAD = NW * KC * CH
NBUF = 2
NLAG = 1
KC2 = 2 * KC
ASPL = 80
MAXC = max(ASPL, KC2 - ASPL)
RPS = NPAD // NS
BLK = 2560
NG = NPAD // BLK
_EPS = 1e-5

_mesh = plsc.VectorSubcoreMesh(core_axis_name="c", subcore_axis_name="s")


def _degree_body(src_hbm, dst_hbm, eye_hbm, z8_hbm, deg_hbm,
                 src_v, dst_v, e0_v, e1_v, deg_sh, sem, sem2):
    cid = lax.axis_index("c")
    sid = lax.axis_index("s")
    wid = sid * NC + cid
    pltpu.sync_copy(src_hbm.at[wid], src_v)
    pltpu.sync_copy(dst_hbm.at[wid], dst_v)
    pltpu.sync_copy(eye_hbm.at[0], e0_v)
    pltpu.sync_copy(eye_hbm.at[1], e1_v)
    pltpu.sync_copy(z8_hbm.at[pl.ds(sid * RPS, RPS)],
                    deg_sh.at[pl.ds(sid * RPS, RPS)])
    plsc.subcore_barrier()

    def body(j, carry):
        pltpu.async_copy(e0_v, deg_sh.at[src_v.at[j]], sem, add=True)
        pltpu.async_copy(e1_v, deg_sh.at[dst_v.at[j]], sem2, add=True)

        @pl.when(j >= 1)
        def _():
            pltpu.make_async_copy(e0_v, deg_sh.at[src_v.at[j - 1]],
                                  sem).wait()
            pltpu.make_async_copy(e1_v, deg_sh.at[dst_v.at[j - 1]],
                                  sem2).wait()

        return carry

    lax.fori_loop(0, KC, body, 0)
    pltpu.make_async_copy(e0_v, deg_sh.at[src_v.at[KC - 1]], sem).wait()
    pltpu.make_async_copy(e1_v, deg_sh.at[dst_v.at[KC - 1]], sem2).wait()
    plsc.subcore_barrier()
    pltpu.sync_copy(deg_sh.at[pl.ds(sid * RPS, RPS)],
                    deg_hbm.at[cid].at[pl.ds(sid * RPS, RPS)])


_degree = functools.partial(
    pl.kernel,
    out_type=jax.ShapeDtypeStruct((NC, NPAD, 8), jnp.float32),
    mesh=_mesh,
    scratch_types=[
        pltpu.VMEM((KC, CH), jnp.int32),
        pltpu.VMEM((KC, CH), jnp.int32),
        pltpu.VMEM((CH, 8), jnp.float32),
        pltpu.VMEM((CH, 8), jnp.float32),
        pltpu.VMEM_SHARED((NPAD, 8), jnp.float32),
        pltpu.SemaphoreType.DMA,
        pltpu.SemaphoreType.DMA,
    ],
    compiler_params=pltpu.CompilerParams(use_tc_tiling_on_sc=False),
)(_degree_body)


def _make_edge_pass(D, nslab):
    def body(*refs):
        xs_list = refs[:nslab]
        src_hbm, dst_hbm, zd_hbm = refs[nslab:nslab + 3]
        out_list = refs[nslab + 3:nslab + 3 + nslab]
        (src_v, dst_v, r0, r1, xs_sh, agg_sh,
         g0, g1, s0, s1) = refs[nslab + 3 + nslab:]
        cid = lax.axis_index("c")
        sid = lax.axis_index("s")
        rows = (r0, r1)
        gsem = (g0, g1)
        ssem = (s0, s1)

        def run(off, cnt, load_idx):
            if load_idx:
                pltpu.sync_copy(src_hbm.at[sid].at[pl.ds(off, cnt)],
                                src_v.at[pl.ds(0, cnt)])
                pltpu.sync_copy(dst_hbm.at[sid].at[pl.ds(off, cnt)],
                                dst_v.at[pl.ds(0, cnt)])
            for b in range(NLAG):
                pltpu.async_copy(xs_sh.at[src_v.at[b]], rows[b], gsem[b])

            def group(gi, carry):
                for b in range(NBUF):
                    j = gi * NBUF + b
                    bh = (b + NLAG) % NBUF
                    pltpu.make_async_copy(xs_sh.at[src_v.at[j]], rows[b],
                                          gsem[b]).wait()
                    pltpu.async_copy(rows[b], agg_sh.at[dst_v.at[j]],
                                     ssem[b], add=True)

                    @pl.when(j >= NLAG)
                    def _():
                        pltpu.make_async_copy(rows[bh],
                                              agg_sh.at[dst_v.at[j - NLAG]],
                                              ssem[bh]).wait()

                    @pl.when(j + NLAG < cnt)
                    def _():
                        pltpu.async_copy(xs_sh.at[src_v.at[j + NLAG]],
                                         rows[bh], gsem[bh])
                return carry

            lax.fori_loop(0, cnt // NBUF, group, 0)
            for c in range(cnt - NLAG, cnt):
                pltpu.make_async_copy(rows[c % NBUF], agg_sh.at[dst_v.at[c]],
                                      ssem[c % NBUF]).wait()

        for k in range(nslab):
            pltpu.sync_copy(xs_list[k].at[pl.ds(sid * RPS, RPS)],
                            xs_sh.at[pl.ds(sid * RPS, RPS)])
            pltpu.sync_copy(zd_hbm.at[pl.ds(sid * RPS, RPS)],
                            agg_sh.at[pl.ds(sid * RPS, RPS)])
            plsc.subcore_barrier()

            @pl.when(cid == 0)
            def _():
                run(0, ASPL, k == 0)

            @pl.when(cid == 1)
            def _():
                run(ASPL, KC2 - ASPL, k == 0)

            plsc.subcore_barrier()
            pltpu.sync_copy(agg_sh.at[pl.ds(sid * RPS, RPS)],
                            out_list[k].at[cid].at[pl.ds(sid * RPS, RPS)])

    return functools.partial(
        pl.kernel,
        out_type=[jax.ShapeDtypeStruct((NC, NPAD, D), jnp.float32)
                  for _ in range(nslab)],
        mesh=_mesh,
        scratch_types=[
            pltpu.VMEM((MAXC, CH), jnp.int32),
            pltpu.VMEM((MAXC, CH), jnp.int32),
        ] + [pltpu.VMEM((CH, D), jnp.float32) for _ in range(NBUF)] + [
            pltpu.VMEM_SHARED((NPAD, D), jnp.float32),
            pltpu.VMEM_SHARED((NPAD, D), jnp.float32),
        ] + [pltpu.SemaphoreType.DMA for _ in range(2 * NBUF)],
        compiler_params=pltpu.CompilerParams(use_tc_tiling_on_sc=False),
    )(body)


_edge64 = _make_edge_pass(64, 1)


def _prep_body(deg_ref, x_ref, no_ref, ni_ref, xsa_ref, xsb_ref):
    d = deg_ref[0] + deg_ref[1]
    do = d[:, 0:1]
    di = d[:, 1:2]
    no = jnp.where(do > 0, lax.rsqrt(jnp.maximum(do, 1.0)), 0.0)
    ni = jnp.where(di > 0, lax.rsqrt(jnp.maximum(di, 1.0)), 0.0)
    no_ref[...] = jnp.broadcast_to(no, (BLK, 8))
    ni_ref[...] = jnp.broadcast_to(ni, (BLK, 8))
    xs = x_ref[...] * no
    xsa_ref[...] = xs[:, :64]
    xsb_ref[...] = xs[:, 64:]


def _prep(degp, xp):
    return pl.pallas_call(
        _prep_body,
        grid=(NG,),
        in_specs=[
            pl.BlockSpec((NC, BLK, 8), lambda i: (0, i, 0)),
            pl.BlockSpec((BLK, 128), lambda i: (i, 0)),
        ],
        out_specs=[
            pl.BlockSpec((BLK, 8), lambda i: (i, 0)),
            pl.BlockSpec((BLK, 8), lambda i: (i, 0)),
            pl.BlockSpec((BLK, 64), lambda i: (i, 0)),
            pl.BlockSpec((BLK, 64), lambda i: (i, 0)),
        ],
        out_shape=[
            jax.ShapeDtypeStruct((NPAD, 8), jnp.float32),
            jax.ShapeDtypeStruct((NPAD, 8), jnp.float32),
            jax.ShapeDtypeStruct((NPAD, 64), jnp.float32),
            jax.ShapeDtypeStruct((NPAD, 64), jnp.float32),
        ],
    )(degp, xp)


def _make_layer_body(slabs):
    def body(*refs):
        p_refs = refs[:slabs]
        (x_ref, ni_ref, no_ref, w_ref, b_ref, rw_ref, rb_ref,
         g_ref, beta_ref, m_ref, v_ref, h_ref, hs_ref) = refs[slabs:]
        ni = ni_ref[...][:, 0:1]
        aggs = [(p[0] + p[1]) * ni for p in p_refs]
        agg = aggs[0] if slabs == 1 else jnp.concatenate(aggs, axis=1)
        hm = jnp.dot(agg, w_ref[...], preferred_element_type=jnp.float32)
        hm = jnp.maximum(hm + b_ref[...], 0.0)
        res = jnp.dot(x_ref[...], rw_ref[...],
                      preferred_element_type=jnp.float32)
        res = jnp.maximum(res + rb_ref[...], 0.0)
        h = hm + res
        h = (h - m_ref[...]) * lax.rsqrt(v_ref[...] + _EPS) * g_ref[...] \
            + beta_ref[...]
        h_ref[...] = h
        hs_ref[...] = h * no_ref[...][:, 0:1]
    return body


def _layer(p_list, x, ni, no, w, b, rw, rb, g, beta, m, v):
    slabs = len(p_list)
    D = 64 * slabs
    vec = lambda: pl.BlockSpec((1, 64), lambda i: (0, 0))
    return pl.pallas_call(
        _make_layer_body(slabs),
        grid=(NG,),
        in_specs=[pl.BlockSpec((NC, BLK, 64), lambda i: (0, i, 0))
                  for _ in range(slabs)] + [
            pl.BlockSpec((BLK, D), lambda i: (i, 0)),
            pl.BlockSpec((BLK, 8), lambda i: (i, 0)),
            pl.BlockSpec((BLK, 8), lambda i: (i, 0)),
            pl.BlockSpec((D, 64), lambda i: (0, 0)),
            vec(),
            pl.BlockSpec((D, 64), lambda i: (0, 0)),
            vec(), vec(), vec(), vec(), vec(),
        ],
        out_specs=[
            pl.BlockSpec((BLK, 64), lambda i: (i, 0)),
            pl.BlockSpec((BLK, 64), lambda i: (i, 0)),
        ],
        out_shape=[
            jax.ShapeDtypeStruct((NPAD, 64), jnp.float32),
            jax.ShapeDtypeStruct((NPAD, 64), jnp.float32),
        ],
    )(*p_list, x, ni, no, w, b.reshape(1, 64), rw, rb.reshape(1, 64),
      g.reshape(1, 64), beta.reshape(1, 64), m.reshape(1, 64),
      v.reshape(1, 64))


def _l3f_body(p_ref, x_ref, ni_ref, gid_ref, w_ref, b_ref, rw_ref, rb_ref,
              g_ref, beta_ref, m_ref, v_ref, w1_ref, b1_ref, w2_ref, b2_ref,
              out_ref, acc_ref):
    i = pl.program_id(0)

    @pl.when(i == 0)
    def _():
        acc_ref[...] = jnp.zeros_like(acc_ref)
        out_ref[...] = jnp.zeros_like(out_ref)

    agg = (p_ref[0] + p_ref[1]) * ni_ref[...][:, 0:1]
    hm = jnp.dot(agg, w_ref[...], preferred_element_type=jnp.float32)
    hm = jnp.maximum(hm + b_ref[...], 0.0)
    res = jnp.dot(x_ref[...], rw_ref[...], preferred_element_type=jnp.float32)
    res = jnp.maximum(res + rb_ref[...], 0.0)
    h = hm + res
    h = (h - m_ref[...]) * lax.rsqrt(v_ref[...] + _EPS) * g_ref[...] \
        + beta_ref[...]
    gid = gid_ref[0, 0]
    oh = (lax.broadcasted_iota(jnp.int32, (G, BLK), 0) == gid[None, :])
    acc_ref[...] += jnp.dot(oh.astype(jnp.float32), h,
                            preferred_element_type=jnp.float32)

    @pl.when(i == NG - 1)
    def _():
        hid = jnp.dot(acc_ref[...], w1_ref[...],
                      preferred_element_type=jnp.float32)
        hid = jnp.maximum(hid + b1_ref[...], 0.0)
        out_ref[...] = jnp.dot(hid, w2_ref[...],
                               preferred_element_type=jnp.float32) \
            + b2_ref[...]


def _layer3_final(p, x, ni, gid3, w, b, rw, rb, g, beta, m, v,
                  w1, b1, w2p, b2p):
    vec = lambda: pl.BlockSpec((1, 64), lambda i: (0, 0))
    return pl.pallas_call(
        _l3f_body,
        grid=(NG,),
        in_specs=[
            pl.BlockSpec((NC, BLK, 64), lambda i: (0, i, 0)),
            pl.BlockSpec((BLK, 64), lambda i: (i, 0)),
            pl.BlockSpec((BLK, 8), lambda i: (i, 0)),
            pl.BlockSpec((1, 1, BLK), lambda i: (i, 0, 0)),
            pl.BlockSpec((64, 64), lambda i: (0, 0)),
            vec(),
            pl.BlockSpec((64, 64), lambda i: (0, 0)),
            vec(), vec(), vec(), vec(), vec(),
            pl.BlockSpec((64, 128), lambda i: (0, 0)),
            pl.BlockSpec((1, 128), lambda i: (0, 0)),
            pl.BlockSpec((128, 128), lambda i: (0, 0)),
            pl.BlockSpec((1, 128), lambda i: (0, 0)),
        ],
        out_specs=pl.BlockSpec((G, 128), lambda i: (0, 0)),
        out_shape=jax.ShapeDtypeStruct((G, 128), jnp.float32),
        scratch_shapes=[pltpu.VMEM((G, 64), jnp.float32)],
    )(p, x, ni, gid3, w, b.reshape(1, 64), rw, rb.reshape(1, 64),
      g.reshape(1, 64), beta.reshape(1, 64), m.reshape(1, 64),
      v.reshape(1, 64), w1, b1, w2p, b2p)


def kernel(node_feats, edge_index, graph_ids,
           W0, b0, Rw0, Rb0, g0, beta0, m0, v0,
           W1, b1, Rw1, Rb1, g1, beta1, m1, v1,
           W2, b2, Rw2, Rb2, g2, beta2, m2, v2,
           Wc1, bc1, Wc2, bc2):
    pad_e = EPAD - E
    srcp = jnp.concatenate(
        [edge_index[0], jnp.full((pad_e,), N, jnp.int32)]).reshape(NW, KC, CH)
    dstp = jnp.concatenate(
        [edge_index[1], jnp.full((pad_e,), N, jnp.int32)]).reshape(NW, KC, CH)
    xp = jnp.pad(node_feats, ((0, NPAD - N), (0, 0)))
    gid3 = jnp.pad(graph_ids, (0, NPAD - N),
                   constant_values=G).reshape(NG, 1, BLK)
    eye = jnp.zeros((2, CH, 8), jnp.float32)
    eye = eye.at[0, :, 0].set(1.0).at[1, :, 1].set(1.0)
    z8 = jnp.zeros((NPAD, 8), jnp.float32)
    z64 = jnp.zeros((NPAD, 64), jnp.float32)

    degp = _degree(srcp, dstp, eye, z8)
    no, ni, xs0a, xs0b = _prep(degp, xp)

    srcE = srcp.reshape(NS, KC2, CH)
    dstE = dstp.reshape(NS, KC2, CH)
    (p0a,) = _edge64(xs0a, srcE, dstE, z64)
    (p0b,) = _edge64(xs0b, srcE, dstE, z64)
    h1, h1s = _layer([p0a, p0b], xp, ni, no,
                     W0, b0, Rw0, Rb0, g0, beta0, m0, v0)

    (p1,) = _edge64(h1s, srcE, dstE, z64)
    h2, h2s = _layer([p1], h1, ni, no, W1, b1, Rw1, Rb1, g1, beta1, m1, v1)

    (p2,) = _edge64(h2s, srcE, dstE, z64)
    w2p = jnp.pad(Wc2, ((0, 0), (0, 126)))
    b2p = jnp.pad(bc2, (0, 126)).reshape(1, 128)
    logits = _layer3_final(p2, h2, ni, gid3, W2, b2, Rw2, Rb2, g2, beta2,
                           m2, v2, Wc1, bc1.reshape(1, 128), w2p, b2p)
    return logits[:, :2]
